# baseline (speedup 1.0000x reference)
import functools

import jax
import jax.numpy as jnp
from jax import lax
from jax.experimental import pallas as pl
from jax.experimental.pallas import tpu as pltpu
from jax.experimental.pallas import tpu_sc as plsc

NUM_NODES = 1024
DIM = 64


def _scores_row_body(sidx_ref, new_ref, w_ref, c_ref, s_ref):
    b = pl.program_id(0)

    @pl.when(new_ref[b] == 1)
    def _():
        w = w_ref[0]
        e = jnp.sum(w * c_ref[...], axis=0, keepdims=True)
        m = jnp.max(e, axis=-1, keepdims=True)
        p = jnp.exp(e - m)
        s_ref[0] = p / jnp.sum(p, axis=-1, keepdims=True)


def _compute_scores_sorted(Wt, c2, sidx, is_new, interpret=False):
    N = NUM_NODES
    B = sidx.shape[0]
    grid_spec = pltpu.PrefetchScalarGridSpec(
        num_scalar_prefetch=2,
        grid=(B,),
        in_specs=[
            pl.BlockSpec((1, DIM, N), lambda b, sref, nref: (sref[b], 0, 0)),
            pl.BlockSpec((DIM, 1), lambda b, sref, nref: (0, 0)),
        ],
        out_specs=pl.BlockSpec((1, 1, N), lambda b, sref, nref: (sref[b], 0, 0)),
    )
    out3 = pl.pallas_call(
        _scores_row_body,
        grid_spec=grid_spec,
        out_shape=jax.ShapeDtypeStruct((N, 1, N), jnp.float32),
        interpret=interpret,
    )(sidx, is_new, Wt, c2)
    return out3.reshape(N, N)


def _make_sc_gather(B, D):
    info = plsc.get_sparse_core_info()
    NC, NS = info.num_cores, info.num_subcores
    NW = NC * NS
    b_per_w = B // NW
    mesh = plsc.VectorSubcoreMesh(core_axis_name="c", subcore_axis_name="s")

    @functools.partial(
        pl.kernel,
        mesh=mesh,
        out_type=jax.ShapeDtypeStruct((B, D), jnp.float32),
        scratch_types=[
            pltpu.VMEM((b_per_w,), jnp.int32),
            pltpu.VMEM((b_per_w, D), jnp.float32),
            pltpu.SemaphoreType.DMA,
        ],
    )
    def gather_k(table_hbm, idx_hbm, out_hbm, idx_v, rows_v, sem):
        wid = lax.axis_index("s") * NC + lax.axis_index("c")
        base = wid * b_per_w
        pltpu.sync_copy(idx_hbm.at[pl.ds(base, b_per_w)], idx_v)
        pltpu.async_copy(table_hbm.at[idx_v], rows_v, sem).wait()
        pltpu.sync_copy(rows_v, out_hbm.at[pl.ds(base, b_per_w)])

    return gather_k


def kernel(node_indices, context_vector, W):
    Wt = W.transpose(0, 2, 1)
    c2 = context_vector.reshape(DIM, 1)
    idx32 = node_indices.astype(jnp.int32)
    sidx = jnp.sort(idx32)
    is_new = jnp.concatenate([
        jnp.ones((1,), jnp.int32),
        (sidx[1:] != sidx[:-1]).astype(jnp.int32),
    ])
    scores = _compute_scores_sorted(Wt, c2, sidx, is_new)
    gather = _make_sc_gather(idx32.shape[0], NUM_NODES)
    return gather(scores, idx32)


# speedup vs baseline: 1.3423x; 1.3423x over previous
import functools

import jax
import jax.numpy as jnp
from jax import lax
from jax.experimental import pallas as pl
from jax.experimental.pallas import tpu as pltpu
from jax.experimental.pallas import tpu_sc as plsc

NUM_NODES = 1024
DIM = 64
BATCH = 1024


def _dedup_body(urows_ref, ucnt_ref, w_hbm, c_ref, s_hbm,
                buf, rowbuf, in_sem, out_sem):
    ucnt = ucnt_ref[0]

    def fetch(k):
        slot = lax.rem(k, 2)
        pltpu.make_async_copy(
            w_hbm.at[urows_ref[k]],
            buf.at[slot],
            in_sem.at[slot],
        ).start()

    @pl.when(ucnt > 0)
    def _():
        fetch(0)

    @pl.when(ucnt > 1)
    def _():
        fetch(1)

    def iter_body(k, _):
        @pl.when(k < ucnt)
        def _():
            slot = lax.rem(k, 2)
            pltpu.make_async_copy(
                w_hbm.at[urows_ref[k]], buf.at[slot], in_sem.at[slot]
            ).wait()
            w = buf[slot]
            e = jnp.sum(w * c_ref[...], axis=0, keepdims=True)
            m = jnp.max(e, axis=-1, keepdims=True)
            p = jnp.exp(e - m)
            r = p / jnp.sum(p, axis=-1, keepdims=True)

            @pl.when(k >= 2)
            def _():
                pltpu.make_async_copy(
                    rowbuf.at[slot], s_hbm.at[urows_ref[k - 2]],
                    out_sem.at[slot],
                ).wait()

            rowbuf[slot] = r
            pltpu.make_async_copy(
                rowbuf.at[slot], s_hbm.at[urows_ref[k]], out_sem.at[slot]
            ).start()

            @pl.when(k + 2 < ucnt)
            def _():
                fetch(k + 2)

        return 0

    lax.fori_loop(0, BATCH, iter_body, 0)

    def drain(k):
        slot = lax.rem(k, 2)
        pltpu.make_async_copy(
            rowbuf.at[slot], s_hbm.at[urows_ref[k]], out_sem.at[slot]
        ).wait()

    @pl.when(ucnt > 1)
    def _():
        drain(ucnt - 2)

    @pl.when(ucnt > 0)
    def _():
        drain(ucnt - 1)


def _compute_scores_dedup(Wt, c2, urows, ucnt, interpret=False):
    N = NUM_NODES
    grid_spec = pltpu.PrefetchScalarGridSpec(
        num_scalar_prefetch=2,
        grid=(1,),
        in_specs=[
            pl.BlockSpec(memory_space=pltpu.MemorySpace.HBM),
            pl.BlockSpec((DIM, 1), lambda i, uref, cref: (0, 0)),
        ],
        out_specs=pl.BlockSpec(memory_space=pltpu.MemorySpace.HBM),
        scratch_shapes=[
            pltpu.VMEM((2, DIM, N), jnp.float32),
            pltpu.VMEM((2, 1, N), jnp.float32),
            pltpu.SemaphoreType.DMA((2,)),
            pltpu.SemaphoreType.DMA((2,)),
        ],
    )
    return pl.pallas_call(
        _dedup_body,
        grid_spec=grid_spec,
        out_shape=jax.ShapeDtypeStruct((N, 1, N), jnp.float32),
        interpret=interpret,
    )(urows, ucnt, Wt, c2)


def _make_sc_gather(B, D):
    info = plsc.get_sparse_core_info()
    NC, NS = info.num_cores, info.num_subcores
    NW = NC * NS
    b_per_w = B // NW
    mesh = plsc.VectorSubcoreMesh(core_axis_name="c", subcore_axis_name="s")

    @functools.partial(
        pl.kernel,
        mesh=mesh,
        out_type=jax.ShapeDtypeStruct((B, D), jnp.float32),
        scratch_types=[
            pltpu.VMEM((b_per_w,), jnp.int32),
            pltpu.VMEM((b_per_w, D), jnp.float32),
            pltpu.SemaphoreType.DMA,
        ],
    )
    def gather_k(table_hbm, idx_hbm, out_hbm, idx_v, rows_v, sem):
        wid = lax.axis_index("s") * NC + lax.axis_index("c")
        base = wid * b_per_w
        pltpu.sync_copy(idx_hbm.at[pl.ds(base, b_per_w)], idx_v)
        pltpu.async_copy(table_hbm.at[idx_v], rows_v, sem).wait()
        pltpu.sync_copy(rows_v, out_hbm.at[pl.ds(base, b_per_w)])

    return gather_k


def kernel(node_indices, context_vector, W):
    Wt = W.transpose(0, 2, 1)
    c2 = context_vector.reshape(DIM, 1)
    idx32 = node_indices.astype(jnp.int32)
    sidx = jnp.sort(idx32)
    is_new = jnp.concatenate([
        jnp.ones((1,), jnp.int32),
        (sidx[1:] != sidx[:-1]).astype(jnp.int32),
    ])
    slots = jnp.cumsum(is_new) - 1
    urows = jnp.zeros((BATCH,), jnp.int32).at[slots].set(sidx)
    ucnt = jnp.sum(is_new, keepdims=True)
    scores3 = _compute_scores_dedup(Wt, c2, urows, ucnt)
    scores = scores3.reshape(NUM_NODES, NUM_NODES)
    gather = _make_sc_gather(BATCH, NUM_NODES)
    return gather(scores, idx32)


# speedup vs baseline: 3.3306x; 2.4813x over previous
import functools

import jax
import jax.numpy as jnp
from jax import lax
from jax.experimental import pallas as pl
from jax.experimental.pallas import tpu as pltpu
from jax.experimental.pallas import tpu_sc as plsc

NUM_NODES = 1024
DIM = 64
BLOCK_M = 64


def _scores_body(w_ref, c_ref, s_ref):
    w = w_ref[...]
    c = c_ref[...]
    e = jnp.sum(w * c[None, :, :], axis=1)
    m = jnp.max(e, axis=-1, keepdims=True)
    p = jnp.exp(e - m)
    s_ref[...] = p / jnp.sum(p, axis=-1, keepdims=True)


def _compute_scores(Wt, c2):
    N = NUM_NODES
    return pl.pallas_call(
        _scores_body,
        grid=(N // BLOCK_M,),
        in_specs=[
            pl.BlockSpec((BLOCK_M, DIM, N), lambda i: (i, 0, 0)),
            pl.BlockSpec((DIM, 1), lambda i: (0, 0)),
        ],
        out_specs=pl.BlockSpec((BLOCK_M, N), lambda i: (i, 0)),
        out_shape=jax.ShapeDtypeStruct((N, N), jnp.float32),
    )(Wt, c2)


def _make_sc_gather(B, D):
    info = plsc.get_sparse_core_info()
    NC, NS = info.num_cores, info.num_subcores
    NW = NC * NS
    b_per_w = B // NW
    mesh = plsc.VectorSubcoreMesh(core_axis_name="c", subcore_axis_name="s")

    @functools.partial(
        pl.kernel,
        mesh=mesh,
        out_type=jax.ShapeDtypeStruct((B, D), jnp.float32),
        scratch_types=[
            pltpu.VMEM((b_per_w,), jnp.int32),
            pltpu.VMEM((b_per_w, D), jnp.float32),
            pltpu.SemaphoreType.DMA,
        ],
    )
    def gather_k(table_hbm, idx_hbm, out_hbm, idx_v, rows_v, sem):
        wid = lax.axis_index("s") * NC + lax.axis_index("c")
        base = wid * b_per_w
        pltpu.sync_copy(idx_hbm.at[pl.ds(base, b_per_w)], idx_v)
        pltpu.async_copy(table_hbm.at[idx_v], rows_v, sem).wait()
        pltpu.sync_copy(rows_v, out_hbm.at[pl.ds(base, b_per_w)])

    return gather_k


def _make_sc_bigread(B, D, chunks):
    info = plsc.get_sparse_core_info()
    NC, NS = info.num_cores, info.num_subcores
    NW = NC * NS
    b_per_w = B // NW
    per_chunk = b_per_w // chunks
    mesh = plsc.VectorSubcoreMesh(core_axis_name="c", subcore_axis_name="s")

    @functools.partial(
        pl.kernel,
        mesh=mesh,
        out_type=jax.ShapeDtypeStruct((B, D), jnp.float32),
        scratch_types=[
            pltpu.VMEM((per_chunk,), jnp.int32),
            pltpu.VMEM((per_chunk, D), jnp.float32),
            pltpu.SemaphoreType.DMA,
        ],
    )
    def big_k(table_hbm, idx_hbm, out_hbm, idx_v, rows_v, sem):
        wid = lax.axis_index("s") * NC + lax.axis_index("c")
        for ch in range(chunks):
            base = wid * b_per_w + ch * per_chunk
            pltpu.sync_copy(idx_hbm.at[pl.ds(base, per_chunk)], idx_v)
            pltpu.async_copy(table_hbm.at[idx_v], rows_v, sem).wait()
            pltpu.sync_copy(rows_v, out_hbm.at[pl.ds(base, per_chunk)])

    return big_k


def kernel(node_indices, context_vector, W):
    Wt = W.transpose(0, 2, 1)
    c2 = context_vector.reshape(DIM, 1)

    Wflat = Wt.reshape(NUM_NODES * DIM, NUM_NODES)
    PB = 16384
    gidx = ((jnp.arange(PB, dtype=jnp.int32) * 37) % (NUM_NODES * DIM))
    dummy = _make_sc_bigread(PB, NUM_NODES, 16)(Wflat, gidx)

    scores = _compute_scores(Wt, c2)
    gather = _make_sc_gather(node_indices.shape[0], NUM_NODES)
    out = gather(scores, node_indices.astype(jnp.int32))
    return out + 0.0 * dummy[:NUM_NODES]


# speedup vs baseline: 5.7097x; 1.7143x over previous
import functools

import jax
import jax.numpy as jnp
from jax import lax
from jax.experimental import pallas as pl
from jax.experimental.pallas import tpu as pltpu
from jax.experimental.pallas import tpu_sc as plsc

NUM_NODES = 1024
DIM = 64
BLOCK_M = 64


def _scores_body(w_ref, c_ref, s_ref):
    w = w_ref[...]
    c = c_ref[...]
    e = jnp.sum(w * c[None, :, :], axis=1)
    m = jnp.max(e, axis=-1, keepdims=True)
    p = jnp.exp(e - m)
    s_ref[...] = p / jnp.sum(p, axis=-1, keepdims=True)


def _compute_scores(Wt, c2):
    N = NUM_NODES
    return pl.pallas_call(
        _scores_body,
        grid=(N // BLOCK_M,),
        in_specs=[
            pl.BlockSpec((BLOCK_M, DIM, N), lambda i: (i, 0, 0)),
            pl.BlockSpec((DIM, 1), lambda i: (0, 0)),
        ],
        out_specs=pl.BlockSpec((BLOCK_M, N), lambda i: (i, 0)),
        out_shape=jax.ShapeDtypeStruct((N, N), jnp.float32),
    )(Wt, c2)


def _make_sc_gather(B, D):
    info = plsc.get_sparse_core_info()
    NC, NS = info.num_cores, info.num_subcores
    NW = NC * NS
    b_per_w = B // NW
    mesh = plsc.VectorSubcoreMesh(core_axis_name="c", subcore_axis_name="s")

    @functools.partial(
        pl.kernel,
        mesh=mesh,
        out_type=jax.ShapeDtypeStruct((B, D), jnp.float32),
        scratch_types=[
            pltpu.VMEM((b_per_w,), jnp.int32),
            pltpu.VMEM((b_per_w, D), jnp.float32),
            pltpu.SemaphoreType.DMA,
        ],
    )
    def gather_k(table_hbm, idx_hbm, out_hbm, idx_v, rows_v, sem):
        wid = lax.axis_index("s") * NC + lax.axis_index("c")
        base = wid * b_per_w
        pltpu.sync_copy(idx_hbm.at[pl.ds(base, b_per_w)], idx_v)
        pltpu.async_copy(table_hbm.at[idx_v], rows_v, sem).wait()
        pltpu.sync_copy(rows_v, out_hbm.at[pl.ds(base, b_per_w)])

    return gather_k


def kernel(node_indices, context_vector, W):
    Wt = W.transpose(0, 2, 1)
    c2 = context_vector.reshape(DIM, 1)
    scores = _compute_scores(Wt, c2)
    gather = _make_sc_gather(node_indices.shape[0], NUM_NODES)
    return gather(scores, node_indices.astype(jnp.int32))


# speedup vs baseline: 5.7800x; 1.0123x over previous
import functools

import jax
import jax.numpy as jnp
from jax import lax
from jax.experimental import pallas as pl
from jax.experimental.pallas import tpu as pltpu
from jax.experimental.pallas import tpu_sc as plsc

NUM_NODES = 1024
DIM = 64
BLOCK_M = 64


def _scores_body(w_ref, c_ref, s_ref):
    w = w_ref[...]
    c = c_ref[0]
    e = jnp.sum(w * c[None, :, None], axis=1)
    m = jnp.max(e, axis=-1, keepdims=True)
    p = jnp.exp(e - m)
    s_ref[...] = p / jnp.sum(p, axis=-1, keepdims=True)


def _compute_scores(Wt, c2):
    N = NUM_NODES
    return pl.pallas_call(
        _scores_body,
        grid=(N // BLOCK_M,),
        in_specs=[
            pl.BlockSpec((BLOCK_M, DIM, N), lambda i: (i, 0, 0)),
            pl.BlockSpec((1, DIM), lambda i: (0, 0)),
        ],
        out_specs=pl.BlockSpec((BLOCK_M, N), lambda i: (i, 0)),
        out_shape=jax.ShapeDtypeStruct((N, N), jnp.float32),
    )(Wt, c2)


def _make_sc_gather(B, D):
    info = plsc.get_sparse_core_info()
    NC, NS = info.num_cores, info.num_subcores
    NW = NC * NS
    b_per_w = B // NW
    mesh = plsc.VectorSubcoreMesh(core_axis_name="c", subcore_axis_name="s")

    @functools.partial(
        pl.kernel,
        mesh=mesh,
        out_type=jax.ShapeDtypeStruct((B, D), jnp.float32),
        scratch_types=[
            pltpu.VMEM((b_per_w,), jnp.int32),
            pltpu.VMEM((b_per_w, D), jnp.float32),
            pltpu.SemaphoreType.DMA,
        ],
    )
    def gather_k(table_hbm, idx_hbm, out_hbm, idx_v, rows_v, sem):
        wid = lax.axis_index("s") * NC + lax.axis_index("c")
        base = wid * b_per_w
        pltpu.sync_copy(idx_hbm.at[pl.ds(base, b_per_w)], idx_v)
        pltpu.async_copy(table_hbm.at[idx_v], rows_v, sem).wait()
        pltpu.sync_copy(rows_v, out_hbm.at[pl.ds(base, b_per_w)])

    return gather_k


def kernel(node_indices, context_vector, W):
    Wt = W.transpose(0, 2, 1)
    c2 = context_vector.reshape(1, DIM)
    scores = _compute_scores(Wt, c2)
    gather = _make_sc_gather(node_indices.shape[0], NUM_NODES)
    return gather(scores, node_indices.astype(jnp.int32))
